# Initial kernel scaffold; baseline (speedup 1.0000x reference)
import jax
import jax.numpy as jnp
from jax.experimental import pallas as pl


def kernel(x, pos_emb):
    raise NotImplementedError("write your pallas kernel here")


# speedup vs baseline: 1.6682x; 1.6682x over previous
import jax
import jax.numpy as jnp
from jax.experimental import pallas as pl

_BT = 1024


def _add_kernel(x_ref, pe_ref, o_ref):
    o_ref[...] = x_ref[...] + pe_ref[...][None]


def kernel(x, pos_emb):
    B, T, C = x.shape
    bt = min(_BT, T)
    grid = (T // bt, B)
    return pl.pallas_call(
        _add_kernel,
        grid=grid,
        in_specs=[
            pl.BlockSpec((1, bt, C), lambda t, b: (b, t, 0)),
            pl.BlockSpec((bt, C), lambda t, b: (t, 0)),
        ],
        out_specs=pl.BlockSpec((1, bt, C), lambda t, b: (b, t, 0)),
        out_shape=jax.ShapeDtypeStruct((B, T, C), x.dtype),
    )(x, pos_emb)


# speedup vs baseline: 1.7397x; 1.0428x over previous
import jax
import jax.numpy as jnp
from jax.experimental import pallas as pl

_BT = 2048


def _add_kernel(x_ref, pe_ref, o_ref):
    o_ref[...] = x_ref[...] + pe_ref[...][None]


def kernel(x, pos_emb):
    B, T, C = x.shape
    bt = min(_BT, T)
    grid = (T // bt, B)
    return pl.pallas_call(
        _add_kernel,
        grid=grid,
        in_specs=[
            pl.BlockSpec((1, bt, C), lambda t, b: (b, t, 0)),
            pl.BlockSpec((bt, C), lambda t, b: (t, 0)),
        ],
        out_specs=pl.BlockSpec((1, bt, C), lambda t, b: (b, t, 0)),
        out_shape=jax.ShapeDtypeStruct((B, T, C), x.dtype),
    )(x, pos_emb)
